# baseline (speedup 1.0000x reference)
import functools

import jax
import jax.numpy as jnp
from jax import lax
from jax.experimental import pallas as pl
from jax.experimental.pallas import tpu as pltpu
from jax.experimental.pallas import tpu_sc as plsc

N = 100000
F = 9
V = 137
D = 128
L = 16
W = D // 2
TW = F * V * W

NW = 32
CHUNK = 160
FC = F * CHUNK
CPW = 20
NPAD = NW * CPW * CHUNK


def _make_sc_call():
    mesh = plsc.VectorSubcoreMesh(core_axis_name="c", subcore_axis_name="s")

    @functools.partial(
        pl.kernel,
        mesh=mesh,
        out_type=jax.ShapeDtypeStruct((NPAD, D), jnp.float32),
        compiler_params=pltpu.CompilerParams(
            needs_layout_passes=False, use_tc_tiling_on_sc=False),
        scratch_types=[
            pltpu.VMEM((TW,), jnp.int32),
            pltpu.VMEM((FC,), jnp.int32),
            pltpu.VMEM((FC,), jnp.int32),
            pltpu.VMEM((CHUNK, D), jnp.float32),
            pltpu.VMEM((CHUNK, D), jnp.float32),
            pltpu.SemaphoreType.DMA,
            pltpu.SemaphoreType.DMA,
            pltpu.SemaphoreType.DMA,
            pltpu.SemaphoreType.DMA,
        ],
    )
    def sc_call(xt_hbm, tab_hbm, out_hbm, tab_v, idx_a, idx_b, out_a, out_b,
                sem_ia, sem_ib, sem_oa, sem_ob):
        cid = lax.axis_index("c")
        sid = lax.axis_index("s")
        wid = sid * 2 + cid
        g0 = wid * CPW

        pltpu.sync_copy(tab_hbm, tab_v)

        iota = lax.iota(jnp.int32, L)

        def issue_idx(g, idx_v, sem):
            pltpu.async_copy(xt_hbm.at[pl.ds(g * FC, FC)], idx_v, sem)

        def wait_idx(g, idx_v, sem):
            pltpu.make_async_copy(xt_hbm.at[pl.ds(g * FC, FC)], idx_v,
                                  sem).wait()

        def issue_out(g, out_v, sem):
            pltpu.async_copy(out_v, out_hbm.at[pl.ds(g * CHUNK, CHUNK)], sem)

        def wait_out(g, out_v, sem):
            pltpu.make_async_copy(out_v, out_hbm.at[pl.ds(g * CHUNK, CHUNK)],
                                  sem).wait()

        def compute(idx_v, out_v):
            def jg_body(jg, carry):
                rbs = [
                    idx_v[pl.ds(f * CHUNK + jg * L, L)] << 6
                    for f in range(F)
                ]

                @plsc.parallel_loop(0, L, unroll=2)
                def _(m):
                    j = jg * L + m
                    mm = jnp.broadcast_to(m, (L,))
                    bases = [
                        rb.at[mm].get(mode="promise_in_bounds") + iota
                        for rb in rbs
                    ]
                    for k in range(W // L):
                        ts = [
                            plsc.bitcast(
                                plsc.load_gather(tab_v, [bases[f] + k * L]),
                                jnp.bfloat16)
                            for f in range(F)
                        ]
                        while len(ts) > 1:
                            nxt = [ts[i] + ts[i + 1]
                                   for i in range(0, len(ts) - 1, 2)]
                            if len(ts) % 2:
                                nxt.append(ts[-1])
                            ts = nxt
                        ev, od = plsc.unpack(
                            ts[0], format=plsc.PackFormat.INTERLEAVED,
                            preferred_element_type=jnp.float32)
                        out_v[j, pl.ds(2 * k * L, L)] = ev
                        out_v[j, pl.ds((2 * k + 1) * L, L)] = od

                return carry

            lax.fori_loop(0, CHUNK // L, jg_body, 0)

        def stage_offsets(idx_v):
            for f in range(1, F):
                for m in range(CHUNK // L):
                    sl = pl.ds(f * CHUNK + m * L, L)
                    idx_v[sl] = idx_v[sl] + f * V

        issue_idx(g0, idx_a, sem_ia)

        def body2(i, carry):
            ga = g0 + 2 * i
            issue_idx(ga + 1, idx_b, sem_ib)
            wait_idx(ga, idx_a, sem_ia)
            stage_offsets(idx_a)

            @pl.when(i > 0)
            def _():
                wait_out(ga, out_a, sem_oa)

            compute(idx_a, out_a)

            @pl.when(i < CPW // 2 - 1)
            def _():
                issue_idx(ga + 2, idx_a, sem_ia)

            issue_out(ga, out_a, sem_oa)
            wait_idx(ga + 1, idx_b, sem_ib)
            stage_offsets(idx_b)

            @pl.when(i > 0)
            def _():
                wait_out(ga + 1, out_b, sem_ob)

            compute(idx_b, out_b)
            issue_out(ga + 1, out_b, sem_ob)
            return carry

        lax.fori_loop(0, CPW // 2, body2, 0)
        wait_out(g0, out_a, sem_oa)
        wait_out(g0, out_b, sem_ob)

    return sc_call


_sc_call = _make_sc_call()


@jax.jit
def kernel(x, tables):
    tb = tables.reshape(F * V, D).astype(jnp.bfloat16)
    tb = tb.reshape(F * V, D // 32, 2, L).transpose(0, 1, 3, 2)
    tabw = jax.lax.bitcast_convert_type(
        tb.reshape(F * V, W, 2), jnp.int32).reshape(TW)

    xt = jnp.pad(x.T, ((0, 0), (0, NPAD - N)))
    xt = xt.reshape(F, NW * CPW, CHUNK).transpose(1, 0, 2).reshape(-1)

    out = _sc_call(xt, tabw)
    return out[:N]


# speedup vs baseline: 1.1311x; 1.1311x over previous
import functools

import jax
import jax.numpy as jnp
from jax import lax
from jax.experimental import pallas as pl
from jax.experimental.pallas import tpu as pltpu
from jax.experimental.pallas import tpu_sc as plsc

N = 100000
F = 9
V = 137
D = 128
L = 16
W = D // 2
TW = F * V * W

NW = 32
AW = N // NW
CHUNK = 160
FC = F * CHUNK
CPW = 20
NFULL = AW // CHUNK
TAIL = AW - NFULL * CHUNK
NPAD = NW * CPW * CHUNK


def _make_sc_call():
    mesh = plsc.VectorSubcoreMesh(core_axis_name="c", subcore_axis_name="s")

    @functools.partial(
        pl.kernel,
        mesh=mesh,
        out_type=jax.ShapeDtypeStruct((N * D,), jnp.float32),
        compiler_params=pltpu.CompilerParams(
            needs_layout_passes=False, use_tc_tiling_on_sc=False),
        scratch_types=[
            pltpu.VMEM((TW,), jnp.int32),
            pltpu.VMEM((FC,), jnp.int32),
            pltpu.VMEM((FC,), jnp.int32),
            pltpu.VMEM((CHUNK * D,), jnp.float32),
            pltpu.VMEM((CHUNK * D,), jnp.float32),
            pltpu.SemaphoreType.DMA,
            pltpu.SemaphoreType.DMA,
            pltpu.SemaphoreType.DMA,
            pltpu.SemaphoreType.DMA,
            pltpu.SemaphoreType.DMA,
        ],
    )
    def sc_call(xt_hbm, tab_hbm, out_hbm, tab_v, idx_a, idx_b, out_a, out_b,
                sem_ia, sem_ib, sem_oa, sem_ob, sem_t):
        cid = lax.axis_index("c")
        sid = lax.axis_index("s")
        wid = sid * 2 + cid
        g0 = wid * CPW
        a0 = wid * AW

        pltpu.async_copy(tab_hbm, tab_v, sem_t)

        iota = lax.iota(jnp.int32, L)

        def issue_idx(g, idx_v, sem):
            pltpu.async_copy(xt_hbm.at[pl.ds(g * FC, FC)], idx_v, sem)

        def wait_idx(g, idx_v, sem):
            pltpu.make_async_copy(xt_hbm.at[pl.ds(g * FC, FC)], idx_v,
                                  sem).wait()

        def issue_out(c, out_v, n_rows, sem):
            pltpu.async_copy(
                out_v.at[pl.ds(0, n_rows * D)],
                out_hbm.at[pl.ds((a0 + c * CHUNK) * D, n_rows * D)], sem)

        def wait_out(c, out_v, n_rows, sem):
            pltpu.make_async_copy(
                out_v.at[pl.ds(0, n_rows * D)],
                out_hbm.at[pl.ds((a0 + c * CHUNK) * D, n_rows * D)],
                sem).wait()

        def stage_offsets(idx_v):
            for f in range(1, F):
                for m in range(CHUNK // L):
                    sl = pl.ds(f * CHUNK + m * L, L)
                    idx_v[sl] = idx_v[sl] + f * V

        def compute(idx_v, out_v):
            def jg_body(jg, carry):
                rbs = [
                    idx_v[pl.ds(f * CHUNK + jg * L, L)] << 6
                    for f in range(F)
                ]

                @plsc.parallel_loop(0, L, unroll=4)
                def _(m):
                    j = jg * L + m
                    mm = jnp.broadcast_to(m, (L,))
                    bases = [
                        rb.at[mm].get(mode="promise_in_bounds") + iota
                        for rb in rbs
                    ]
                    for k in range(W // L):
                        ts = [
                            plsc.bitcast(
                                plsc.load_gather(tab_v, [bases[f] + k * L]),
                                jnp.bfloat16)
                            for f in range(F)
                        ]
                        while len(ts) > 1:
                            nxt = [ts[i] + ts[i + 1]
                                   for i in range(0, len(ts) - 1, 2)]
                            if len(ts) % 2:
                                nxt.append(ts[-1])
                            ts = nxt
                        ev, od = plsc.unpack(
                            ts[0], format=plsc.PackFormat.INTERLEAVED,
                            preferred_element_type=jnp.float32)
                        out_v[pl.ds(j * D + 2 * k * L, L)] = ev
                        out_v[pl.ds(j * D + (2 * k + 1) * L, L)] = od

                return carry

            lax.fori_loop(0, CHUNK // L, jg_body, 0)

        issue_idx(g0, idx_a, sem_ia)
        pltpu.make_async_copy(tab_hbm, tab_v, sem_t).wait()

        NPAIR = (NFULL - 1) // 2

        def body2(i, carry):
            ca = 2 * i
            issue_idx(g0 + ca + 1, idx_b, sem_ib)
            wait_idx(g0 + ca, idx_a, sem_ia)
            stage_offsets(idx_a)

            @pl.when(i > 0)
            def _():
                wait_out(ca, out_a, CHUNK, sem_oa)

            compute(idx_a, out_a)
            issue_idx(g0 + ca + 2, idx_a, sem_ia)
            issue_out(ca, out_a, CHUNK, sem_oa)
            wait_idx(g0 + ca + 1, idx_b, sem_ib)
            stage_offsets(idx_b)

            @pl.when(i > 0)
            def _():
                wait_out(ca + 1, out_b, CHUNK, sem_ob)

            compute(idx_b, out_b)
            issue_out(ca + 1, out_b, CHUNK, sem_ob)
            return carry

        lax.fori_loop(0, NPAIR, body2, 0)

        c18 = NFULL - 1
        issue_idx(g0 + c18 + 1, idx_b, sem_ib)
        wait_idx(g0 + c18, idx_a, sem_ia)
        stage_offsets(idx_a)
        wait_out(c18, out_a, CHUNK, sem_oa)
        compute(idx_a, out_a)
        issue_out(c18, out_a, CHUNK, sem_oa)
        wait_idx(g0 + c18 + 1, idx_b, sem_ib)
        stage_offsets(idx_b)
        wait_out(c18 + 1, out_b, CHUNK, sem_ob)
        compute(idx_b, out_b)
        issue_out(c18 + 1, out_b, TAIL, sem_ob)
        wait_out(c18, out_a, CHUNK, sem_oa)
        wait_out(c18 + 1, out_b, TAIL, sem_ob)

    return sc_call


_sc_call = _make_sc_call()


@jax.jit
def kernel(x, tables):
    tb = tables.reshape(F * V, D).astype(jnp.bfloat16)
    tb = tb.reshape(F * V, D // 32, 2, L).transpose(0, 1, 3, 2)
    tabw = jax.lax.bitcast_convert_type(
        tb.reshape(F * V, W, 2), jnp.int32).reshape(TW)

    xt = x.T.reshape(F, NW, AW)
    xt = jnp.pad(xt, ((0, 0), (0, 0), (0, CPW * CHUNK - AW)))
    xt = xt.reshape(F, NW * CPW, CHUNK).transpose(1, 0, 2).reshape(-1)

    out = _sc_call(xt, tabw)
    return out.reshape(N, D)
